# baseline (speedup 1.0000x reference)
import functools

import jax
import jax.numpy as jnp
from jax import lax
from jax.experimental import pallas as pl
from jax.experimental.pallas import tpu as pltpu
from jax.experimental.pallas import tpu_sc as plsc

VOCAB = 1000000
EMBED_DIM = 64
MAXLEN = 200
BATCH = 4096
OUTPUT_SIZE = 10

NUM_CORES = 2
NUM_SUBCORES = 16
NUM_WORKERS = NUM_CORES * NUM_SUBCORES
ROWS_PER_WORKER = BATCH // NUM_WORKERS
CHUNK_A = 128
CHUNK_B = MAXLEN - CHUNK_A
LANES = 16
MAXLEN_PAD = 256
VECS_PER_ROW = EMBED_DIM // LANES
UNROLL = 4


def _pool_body(idx_hbm, table_hbm, out_hbm, idx_v, rows_v, pooled_v, sem0, sem1):
    wid = lax.axis_index("s") * NUM_CORES + lax.axis_index("c")
    rbase = wid * ROWS_PER_WORKER

    pltpu.sync_copy(idx_hbm.at[pl.ds(rbase, ROWS_PER_WORKER), :], idx_v)

    sems = (sem0, sem1)

    def issue(r, slot):
        pltpu.async_copy(table_hbm.at[idx_v.at[r, pl.ds(0, CHUNK_A)]],
                         rows_v.at[slot, pl.ds(0, CHUNK_A)], sems[slot])
        pltpu.async_copy(table_hbm.at[idx_v.at[r, pl.ds(CHUNK_A, CHUNK_B)]],
                         rows_v.at[slot, pl.ds(CHUNK_A, CHUNK_B)], sems[slot])

    def wait_slot(slot):
        pltpu.make_async_copy(table_hbm.at[pl.ds(0, MAXLEN), :],
                              rows_v.at[slot], sems[slot]).wait()

    def accum(r, slot):
        def step(i, acc):
            base = i * UNROLL
            acc = list(acc)
            for u in range(UNROLL):
                row = base + u
                for d in range(VECS_PER_ROW):
                    acc[d] = acc[d] + rows_v[slot, row, pl.ds(d * LANES, LANES)]
            return tuple(acc)

        zero = jnp.zeros((LANES,), jnp.float32)
        acc = lax.fori_loop(0, MAXLEN // UNROLL, step, (zero,) * VECS_PER_ROW)
        scale = jnp.float32(1.0 / MAXLEN)
        for d in range(VECS_PER_ROW):
            pooled_v[r, pl.ds(d * LANES, LANES)] = acc[d] * scale

    issue(0, 0)
    issue(1, 1)

    def outer(i, carry):
        r0 = 2 * i
        for slot in range(2):
            r = r0 + slot
            wait_slot(slot)
            accum(r, slot)

            @pl.when(r + 2 < ROWS_PER_WORKER)
            def _():
                issue(r + 2, slot)
        return carry

    lax.fori_loop(0, ROWS_PER_WORKER // 2, outer, 0)

    pltpu.sync_copy(pooled_v,
                    out_hbm.at[pl.ds(wid * ROWS_PER_WORKER, ROWS_PER_WORKER), :])


_pool_call = pl.kernel(
    _pool_body,
    out_type=jax.ShapeDtypeStruct((BATCH, EMBED_DIM), jnp.float32),
    mesh=plsc.VectorSubcoreMesh(core_axis_name="c", subcore_axis_name="s",
                                num_cores=NUM_CORES, num_subcores=NUM_SUBCORES),
    scratch_types=[
        pltpu.VMEM((ROWS_PER_WORKER, MAXLEN_PAD), jnp.int32),
        pltpu.VMEM((2, MAXLEN, EMBED_DIM), jnp.float32),
        pltpu.VMEM((ROWS_PER_WORKER, EMBED_DIM), jnp.float32),
        pltpu.SemaphoreType.DMA,
        pltpu.SemaphoreType.DMA,
    ],
    compiler_params=pltpu.CompilerParams(use_tc_tiling_on_sc=False),
)


def _dense_softmax_body(pooled_ref, w_ref, b_ref, out_ref):
    logits = jnp.dot(pooled_ref[...], w_ref[...],
                     preferred_element_type=jnp.float32) + b_ref[...]
    m = jnp.max(logits, axis=-1, keepdims=True)
    e = jnp.exp(logits - m)
    out_ref[...] = e / jnp.sum(e, axis=-1, keepdims=True)


_dense_call = pl.pallas_call(
    _dense_softmax_body,
    out_shape=jax.ShapeDtypeStruct((BATCH, 128), jnp.float32),
)


def _transpose_pad_body(idxt_ref, out_ref):
    x = idxt_ref[...].T
    out_ref[...] = jnp.concatenate(
        [x, jnp.zeros((BATCH, MAXLEN_PAD - MAXLEN), jnp.int32)], axis=1)


_transpose_call = pl.pallas_call(
    _transpose_pad_body,
    out_shape=jax.ShapeDtypeStruct((BATCH, MAXLEN_PAD), jnp.int32),
)


def kernel(indices, table, W, b):
    idx_pad = _transpose_call(indices.astype(jnp.int32).T)
    pooled = _pool_call(idx_pad, table)
    w_pad = jnp.zeros((EMBED_DIM, 128), jnp.float32).at[:, :OUTPUT_SIZE].set(W)
    b_pad = jnp.full((1, 128), -1e30, jnp.float32).at[0, :OUTPUT_SIZE].set(b)
    probs_pad = _dense_call(pooled, w_pad, b_pad)
    return probs_pad[:, :OUTPUT_SIZE]


# speedup vs baseline: 1.0027x; 1.0027x over previous
import functools

import jax
import jax.numpy as jnp
from jax import lax
from jax.experimental import pallas as pl
from jax.experimental.pallas import tpu as pltpu
from jax.experimental.pallas import tpu_sc as plsc

VOCAB = 1000000
EMBED_DIM = 64
MAXLEN = 200
BATCH = 4096
OUTPUT_SIZE = 10

NUM_CORES = 2
NUM_SUBCORES = 16
NUM_WORKERS = NUM_CORES * NUM_SUBCORES
ROWS_PER_WORKER = BATCH // NUM_WORKERS
CHUNK_A = 128
CHUNK_B = MAXLEN - CHUNK_A
LANES = 16
MAXLEN_PAD = 256
VECS_PER_ROW = EMBED_DIM // LANES
UNROLL = 4


def _pool_body(idx_hbm, table_hbm, out_hbm, idx_v, rows_v, pooled_v, sem0, sem1):
    wid = lax.axis_index("s") * NUM_CORES + lax.axis_index("c")
    rbase = wid * ROWS_PER_WORKER

    pltpu.sync_copy(idx_hbm.at[pl.ds(rbase * MAXLEN, ROWS_PER_WORKER * MAXLEN)],
                    idx_v)

    sems = (sem0, sem1)

    def issue(r, slot):
        pltpu.async_copy(table_hbm.at[idx_v.at[pl.ds(r * MAXLEN, CHUNK_A)]],
                         rows_v.at[slot, pl.ds(0, CHUNK_A)], sems[slot])
        pltpu.async_copy(table_hbm.at[idx_v.at[pl.ds(r * MAXLEN + CHUNK_A, CHUNK_B)]],
                         rows_v.at[slot, pl.ds(CHUNK_A, CHUNK_B)], sems[slot])

    def wait_slot(slot):
        pltpu.make_async_copy(table_hbm.at[pl.ds(0, MAXLEN), :],
                              rows_v.at[slot], sems[slot]).wait()

    def accum(r, slot):
        def step(i, acc):
            base = i * UNROLL
            acc = list(acc)
            for u in range(UNROLL):
                row = base + u
                for d in range(VECS_PER_ROW):
                    acc[d] = acc[d] + rows_v[slot, row, pl.ds(d * LANES, LANES)]
            return tuple(acc)

        zero = jnp.zeros((LANES,), jnp.float32)
        acc = lax.fori_loop(0, MAXLEN // UNROLL, step, (zero,) * VECS_PER_ROW)
        scale = jnp.float32(1.0 / MAXLEN)
        for d in range(VECS_PER_ROW):
            pooled_v[r, pl.ds(d * LANES, LANES)] = acc[d] * scale

    issue(0, 0)
    issue(1, 1)

    def outer(i, carry):
        r0 = 2 * i
        for slot in range(2):
            r = r0 + slot
            wait_slot(slot)
            accum(r, slot)

            @pl.when(r + 2 < ROWS_PER_WORKER)
            def _():
                issue(r + 2, slot)
        return carry

    lax.fori_loop(0, ROWS_PER_WORKER // 2, outer, 0)

    pltpu.sync_copy(pooled_v,
                    out_hbm.at[pl.ds(wid * ROWS_PER_WORKER, ROWS_PER_WORKER), :])


_pool_call = pl.kernel(
    _pool_body,
    out_type=jax.ShapeDtypeStruct((BATCH, EMBED_DIM), jnp.float32),
    mesh=plsc.VectorSubcoreMesh(core_axis_name="c", subcore_axis_name="s",
                                num_cores=NUM_CORES, num_subcores=NUM_SUBCORES),
    scratch_types=[
        pltpu.VMEM((ROWS_PER_WORKER * MAXLEN,), jnp.int32),
        pltpu.VMEM((2, MAXLEN, EMBED_DIM), jnp.float32),
        pltpu.VMEM((ROWS_PER_WORKER, EMBED_DIM), jnp.float32),
        pltpu.SemaphoreType.DMA,
        pltpu.SemaphoreType.DMA,
    ],
    compiler_params=pltpu.CompilerParams(use_tc_tiling_on_sc=False),
)


def _dense_softmax_body(pooled_ref, w_ref, b_ref, out_ref):
    logits = jnp.dot(pooled_ref[...], w_ref[...],
                     preferred_element_type=jnp.float32) + b_ref[...]
    m = jnp.max(logits, axis=-1, keepdims=True)
    e = jnp.exp(logits - m)
    out_ref[...] = e / jnp.sum(e, axis=-1, keepdims=True)


_dense_call = pl.pallas_call(
    _dense_softmax_body,
    out_shape=jax.ShapeDtypeStruct((BATCH, 128), jnp.float32),
)


def kernel(indices, table, W, b):
    idx_flat = jnp.clip(indices.astype(jnp.int32).reshape(-1), 0, VOCAB - 1)
    pooled = _pool_call(idx_flat, table)
    w_pad = jnp.zeros((EMBED_DIM, 128), jnp.float32).at[:, :OUTPUT_SIZE].set(W)
    b_pad = jnp.full((1, 128), -1e30, jnp.float32).at[0, :OUTPUT_SIZE].set(b)
    probs_pad = _dense_call(pooled, w_pad, b_pad)
    return probs_pad[:, :OUTPUT_SIZE]


# speedup vs baseline: 1.5504x; 1.5462x over previous
import functools

import jax
import jax.numpy as jnp
from jax import lax
from jax.experimental import pallas as pl
from jax.experimental.pallas import tpu as pltpu
from jax.experimental.pallas import tpu_sc as plsc

VOCAB = 1000000
EMBED_DIM = 64
MAXLEN = 200
BATCH = 4096
OUTPUT_SIZE = 10

NUM_CORES = 2
NUM_SUBCORES = 16
NUM_WORKERS = NUM_CORES * NUM_SUBCORES
ROWS_PER_WORKER = BATCH // NUM_WORKERS
CHUNK_A = 128
CHUNK_B = MAXLEN - CHUNK_A
LANES = 16
MAXLEN_PAD = 256
VECS_PER_ROW = EMBED_DIM // LANES
UNROLL = 4


def _pool_body(idx_hbm, table_hbm, out_hbm, idx_v, rows_v, pooled_v, sem0, sem1):
    wid = lax.axis_index("s") * NUM_CORES + lax.axis_index("c")
    rbase = wid * ROWS_PER_WORKER

    pltpu.sync_copy(idx_hbm.at[pl.ds(rbase * MAXLEN, ROWS_PER_WORKER * MAXLEN)],
                    idx_v)

    sems = (sem0, sem1)

    def issue(r, slot):
        pltpu.async_copy(table_hbm.at[idx_v.at[pl.ds(r * MAXLEN, CHUNK_A)]],
                         rows_v.at[slot, pl.ds(0, CHUNK_A)], sems[slot])
        pltpu.async_copy(table_hbm.at[idx_v.at[pl.ds(r * MAXLEN + CHUNK_A, CHUNK_B)]],
                         rows_v.at[slot, pl.ds(CHUNK_A, CHUNK_B)], sems[slot])

    def wait_slot(slot):
        pltpu.make_async_copy(table_hbm.at[pl.ds(0, MAXLEN), :],
                              rows_v.at[slot], sems[slot]).wait()

    def accum(r, slot):
        def step(i, acc):
            base = i * UNROLL
            acc = list(acc)
            for u in range(UNROLL):
                row = base + u
                for d in range(VECS_PER_ROW):
                    acc[d] = acc[d] + rows_v[slot, row, pl.ds(d * LANES, LANES)]
            return tuple(acc)

        zero = jnp.zeros((LANES,), jnp.float32)
        acc = lax.fori_loop(0, MAXLEN // UNROLL, step, (zero,) * VECS_PER_ROW)
        scale = jnp.float32(1.0 / MAXLEN)
        for d in range(VECS_PER_ROW):
            pooled_v[r, pl.ds(d * LANES, LANES)] = acc[d] * scale

    issue(0, 0)
    issue(1, 1)

    def outer(i, carry):
        r0 = 2 * i
        for slot in range(2):
            r = r0 + slot
            wait_slot(slot)
            accum(r, slot)

            @pl.when(r + 2 < ROWS_PER_WORKER)
            def _():
                issue(r + 2, slot)
        return carry

    lax.fori_loop(0, ROWS_PER_WORKER // 2, outer, 0)

    pltpu.sync_copy(pooled_v,
                    out_hbm.at[pl.ds(wid * ROWS_PER_WORKER, ROWS_PER_WORKER), :])


_pool_call = pl.kernel(
    _pool_body,
    out_type=jax.ShapeDtypeStruct((BATCH, EMBED_DIM), jnp.float32),
    mesh=plsc.VectorSubcoreMesh(core_axis_name="c", subcore_axis_name="s",
                                num_cores=NUM_CORES, num_subcores=NUM_SUBCORES),
    scratch_types=[
        pltpu.VMEM((ROWS_PER_WORKER * MAXLEN,), jnp.int32),
        pltpu.VMEM((2, MAXLEN, EMBED_DIM), jnp.float32),
        pltpu.VMEM((ROWS_PER_WORKER, EMBED_DIM), jnp.float32),
        pltpu.SemaphoreType.DMA,
        pltpu.SemaphoreType.DMA,
    ],
    compiler_params=pltpu.CompilerParams(use_tc_tiling_on_sc=False),
)


def _dense_softmax_body(pooled_ref, w_ref, b_ref, out_ref):
    logits = jnp.dot(pooled_ref[...], w_ref[...],
                     preferred_element_type=jnp.float32) + b_ref[...]
    m = jnp.max(logits, axis=-1, keepdims=True)
    e = jnp.exp(logits - m)
    out_ref[...] = e / jnp.sum(e, axis=-1, keepdims=True)


_dense_call = pl.pallas_call(
    _dense_softmax_body,
    out_shape=jax.ShapeDtypeStruct((BATCH, 128), jnp.float32),
)


_RELAYOUT_COLS = 4096
_RELAYOUT_HALF = _RELAYOUT_COLS // 2
_RELAYOUT_GRID = -(-VOCAB // _RELAYOUT_COLS)
VOCAB_LIN = _RELAYOUT_GRID * _RELAYOUT_COLS


def _table_relayout_body(tt_ref, out_ref):
    x = tt_ref[...]
    out_ref[...] = jnp.concatenate(
        [x[:, :_RELAYOUT_HALF].T, x[:, _RELAYOUT_HALF:].T], axis=1)


_table_relayout_call = pl.pallas_call(
    _table_relayout_body,
    grid=(_RELAYOUT_GRID,),
    in_specs=[pl.BlockSpec((EMBED_DIM, _RELAYOUT_COLS), lambda i: (0, i))],
    out_specs=pl.BlockSpec((_RELAYOUT_HALF, 128), lambda i: (i, 0)),
    out_shape=jax.ShapeDtypeStruct((VOCAB_LIN // 2, 128), jnp.float32),
)


def kernel(indices, table, W, b):
    v = jnp.clip(indices.astype(jnp.int32).reshape(-1), 0, VOCAB - 1)
    i_blk = v >> 12
    r_off = v & (_RELAYOUT_COLS - 1)
    idx_flat = (((i_blk << 11) | (r_off & (_RELAYOUT_HALF - 1))) << 1) | (r_off >> 11)
    table_lin = _table_relayout_call(table.T).reshape(VOCAB_LIN, EMBED_DIM)
    pooled = _pool_call(idx_flat, table_lin)
    w_pad = jnp.zeros((EMBED_DIM, 128), jnp.float32).at[:, :OUTPUT_SIZE].set(W)
    b_pad = jnp.full((1, 128), -1e30, jnp.float32).at[0, :OUTPUT_SIZE].set(b)
    probs_pad = _dense_call(pooled, w_pad, b_pad)
    return probs_pad[:, :OUTPUT_SIZE]


# speedup vs baseline: 1.8089x; 1.1668x over previous
import functools

import jax
import jax.numpy as jnp
from jax import lax
from jax.experimental import pallas as pl
from jax.experimental.pallas import tpu as pltpu
from jax.experimental.pallas import tpu_sc as plsc

VOCAB = 1000000
EMBED_DIM = 64
MAXLEN = 200
BATCH = 4096
OUTPUT_SIZE = 10

NUM_CORES = 2
NUM_SUBCORES = 16
NUM_WORKERS = NUM_CORES * NUM_SUBCORES
ROWS_PER_WORKER = BATCH // NUM_WORKERS
CHUNK_A = 128
CHUNK_B = MAXLEN - CHUNK_A
LANES = 16
MAXLEN_PAD = 256
VECS_PER_ROW = EMBED_DIM // LANES
UNROLL = 4


def _pool_body(idx_hbm, table_hbm, out_hbm, idx_v, rows_v, pooled_v, sem0, sem1):
    wid = lax.axis_index("s") * NUM_CORES + lax.axis_index("c")
    rbase = wid * ROWS_PER_WORKER

    pltpu.sync_copy(idx_hbm.at[pl.ds(rbase * MAXLEN, ROWS_PER_WORKER * MAXLEN)],
                    idx_v)

    sems = (sem0, sem1)

    def issue(r, slot):
        pltpu.async_copy(table_hbm.at[idx_v.at[pl.ds(r * MAXLEN, CHUNK_A)]],
                         rows_v.at[slot, pl.ds(0, CHUNK_A)], sems[slot])
        pltpu.async_copy(table_hbm.at[idx_v.at[pl.ds(r * MAXLEN + CHUNK_A, CHUNK_B)]],
                         rows_v.at[slot, pl.ds(CHUNK_A, CHUNK_B)], sems[slot])

    def wait_slot(slot):
        pltpu.make_async_copy(table_hbm.at[pl.ds(0, MAXLEN), :],
                              rows_v.at[slot], sems[slot]).wait()

    def accum(r, slot):
        def step(i, acc):
            base = i * UNROLL
            acc = list(acc)
            for u in range(UNROLL):
                row = base + u
                for d in range(VECS_PER_ROW):
                    acc[d] = acc[d] + rows_v[slot, row, pl.ds(d * LANES, LANES)]
            return tuple(acc)

        zero = jnp.zeros((LANES,), jnp.float32)
        acc = lax.fori_loop(0, MAXLEN // UNROLL, step, (zero,) * VECS_PER_ROW)
        scale = jnp.float32(1.0 / MAXLEN)
        for d in range(VECS_PER_ROW):
            pooled_v[r, pl.ds(d * LANES, LANES)] = acc[d] * scale

    issue(0, 0)
    issue(1, 1)

    def outer(i, carry):
        r0 = 2 * i
        for slot in range(2):
            r = r0 + slot
            wait_slot(slot)
            accum(r, slot)

            @pl.when(r + 2 < ROWS_PER_WORKER)
            def _():
                issue(r + 2, slot)
        return carry

    lax.fori_loop(0, ROWS_PER_WORKER // 2, outer, 0)

    pltpu.sync_copy(pooled_v,
                    out_hbm.at[pl.ds(wid * ROWS_PER_WORKER, ROWS_PER_WORKER), :])


_pool_call = pl.kernel(
    _pool_body,
    out_type=jax.ShapeDtypeStruct((BATCH, EMBED_DIM), jnp.float32),
    mesh=plsc.VectorSubcoreMesh(core_axis_name="c", subcore_axis_name="s",
                                num_cores=NUM_CORES, num_subcores=NUM_SUBCORES),
    scratch_types=[
        pltpu.VMEM((ROWS_PER_WORKER * MAXLEN,), jnp.int32),
        pltpu.VMEM((2, MAXLEN, EMBED_DIM), jnp.float32),
        pltpu.VMEM((ROWS_PER_WORKER, EMBED_DIM), jnp.float32),
        pltpu.SemaphoreType.DMA,
        pltpu.SemaphoreType.DMA,
    ],
    compiler_params=pltpu.CompilerParams(use_tc_tiling_on_sc=False),
)


def _dense_softmax_body(pooled_ref, w_ref, b_ref, out_ref):
    logits = jnp.dot(pooled_ref[...], w_ref[...],
                     preferred_element_type=jnp.float32) + b_ref[...]
    m = jnp.max(logits, axis=-1, keepdims=True)
    e = jnp.exp(logits - m)
    out_ref[...] = e / jnp.sum(e, axis=-1, keepdims=True)


_dense_call = pl.pallas_call(
    _dense_softmax_body,
    out_shape=jax.ShapeDtypeStruct((BATCH, 128), jnp.float32),
)


_RELAYOUT_COLS = 8192
_RELAYOUT_HALF = _RELAYOUT_COLS // 2
_RELAYOUT_SHIFT = 13
_RELAYOUT_GRID = -(-VOCAB // _RELAYOUT_COLS)
VOCAB_LIN = _RELAYOUT_GRID * _RELAYOUT_COLS


def _table_relayout_body(tt_ref, out_ref):
    out_ref[:, :EMBED_DIM] = tt_ref[:, :_RELAYOUT_HALF].T
    out_ref[:, EMBED_DIM:] = tt_ref[:, _RELAYOUT_HALF:].T


_table_relayout_call = pl.pallas_call(
    _table_relayout_body,
    grid=(_RELAYOUT_GRID,),
    in_specs=[pl.BlockSpec((EMBED_DIM, _RELAYOUT_COLS), lambda i: (0, i))],
    out_specs=pl.BlockSpec((_RELAYOUT_HALF, 128), lambda i: (i, 0)),
    out_shape=jax.ShapeDtypeStruct((VOCAB_LIN // 2, 128), jnp.float32),
)


def kernel(indices, table, W, b):
    v = jnp.clip(indices.astype(jnp.int32).reshape(-1), 0, VOCAB - 1)
    i_blk = v >> _RELAYOUT_SHIFT
    r_off = v & (_RELAYOUT_COLS - 1)
    idx_flat = ((((i_blk << (_RELAYOUT_SHIFT - 1)) | (r_off & (_RELAYOUT_HALF - 1)))
                 << 1) | (r_off >> (_RELAYOUT_SHIFT - 1)))
    table_lin = _table_relayout_call(table.T).reshape(VOCAB_LIN, EMBED_DIM)
    pooled = _pool_call(idx_flat, table_lin)
    w_pad = jnp.zeros((EMBED_DIM, 128), jnp.float32).at[:, :OUTPUT_SIZE].set(W)
    b_pad = jnp.full((1, 128), -1e30, jnp.float32).at[0, :OUTPUT_SIZE].set(b)
    probs_pad = _dense_call(pooled, w_pad, b_pad)
    return probs_pad[:, :OUTPUT_SIZE]


# speedup vs baseline: 1.9612x; 1.0842x over previous
import functools

import jax
import jax.numpy as jnp
from jax import lax
from jax.experimental import pallas as pl
from jax.experimental.pallas import tpu as pltpu
from jax.experimental.pallas import tpu_sc as plsc

VOCAB = 1000000
EMBED_DIM = 64
MAXLEN = 200
BATCH = 4096
OUTPUT_SIZE = 10

NUM_CORES = 2
NUM_SUBCORES = 16
NUM_WORKERS = NUM_CORES * NUM_SUBCORES
ROWS_PER_WORKER = BATCH // NUM_WORKERS
CHUNK_A = 128
CHUNK_B = MAXLEN - CHUNK_A
LANES = 16
MAXLEN_PAD = 256
VECS_PER_ROW = EMBED_DIM // LANES
UNROLL = 4


def _pool_body(idx_hbm, table_hbm, out_hbm, idx_v, rows_v, pooled_v, sem0, sem1):
    wid = lax.axis_index("s") * NUM_CORES + lax.axis_index("c")
    rbase = wid * ROWS_PER_WORKER

    pltpu.sync_copy(idx_hbm.at[pl.ds(rbase * MAXLEN, ROWS_PER_WORKER * MAXLEN)],
                    idx_v)

    sems = (sem0, sem1)

    def issue(r, slot):
        pltpu.async_copy(table_hbm.at[idx_v.at[pl.ds(r * MAXLEN, CHUNK_A)]],
                         rows_v.at[slot, pl.ds(0, CHUNK_A)], sems[slot])
        pltpu.async_copy(table_hbm.at[idx_v.at[pl.ds(r * MAXLEN + CHUNK_A, CHUNK_B)]],
                         rows_v.at[slot, pl.ds(CHUNK_A, CHUNK_B)], sems[slot])

    def wait_slot(slot):
        pltpu.make_async_copy(table_hbm.at[pl.ds(0, MAXLEN), :],
                              rows_v.at[slot], sems[slot]).wait()

    def accum(r, slot):
        def step(i, acc):
            base = i * UNROLL
            acc = list(acc)
            for u in range(UNROLL):
                row = base + u
                for d in range(VECS_PER_ROW):
                    acc[d] = acc[d] + rows_v[slot, row, pl.ds(d * LANES, LANES)]
            return tuple(acc)

        zero = jnp.zeros((LANES,), jnp.float32)
        acc = lax.fori_loop(0, MAXLEN // UNROLL, step, (zero,) * VECS_PER_ROW)
        scale = jnp.float32(1.0 / MAXLEN)
        for d in range(VECS_PER_ROW):
            pooled_v[r, pl.ds(d * LANES, LANES)] = acc[d] * scale

    issue(0, 0)
    issue(1, 1)

    def outer(i, carry):
        r0 = 2 * i
        for slot in range(2):
            r = r0 + slot
            wait_slot(slot)
            accum(r, slot)

            @pl.when(r + 2 < ROWS_PER_WORKER)
            def _():
                issue(r + 2, slot)
        return carry

    lax.fori_loop(0, ROWS_PER_WORKER // 2, outer, 0)

    pltpu.sync_copy(pooled_v,
                    out_hbm.at[pl.ds(wid * ROWS_PER_WORKER, ROWS_PER_WORKER), :])


_pool_call = pl.kernel(
    _pool_body,
    out_type=jax.ShapeDtypeStruct((BATCH, EMBED_DIM), jnp.float32),
    mesh=plsc.VectorSubcoreMesh(core_axis_name="c", subcore_axis_name="s",
                                num_cores=NUM_CORES, num_subcores=NUM_SUBCORES),
    scratch_types=[
        pltpu.VMEM((ROWS_PER_WORKER * MAXLEN,), jnp.int32),
        pltpu.VMEM((2, MAXLEN, EMBED_DIM), jnp.float32),
        pltpu.VMEM((ROWS_PER_WORKER, EMBED_DIM), jnp.float32),
        pltpu.SemaphoreType.DMA,
        pltpu.SemaphoreType.DMA,
    ],
    compiler_params=pltpu.CompilerParams(use_tc_tiling_on_sc=False),
)


def _dense_softmax_body(pooled_ref, w_ref, b_ref, out_ref):
    logits = jnp.dot(pooled_ref[...], w_ref[...],
                     preferred_element_type=jnp.float32) + b_ref[...]
    m = jnp.max(logits, axis=-1, keepdims=True)
    e = jnp.exp(logits - m)
    out_ref[...] = e / jnp.sum(e, axis=-1, keepdims=True)


_dense_call = pl.pallas_call(
    _dense_softmax_body,
    out_shape=jax.ShapeDtypeStruct((BATCH, 128), jnp.float32),
)


_RELAYOUT_COLS = 16384
_RELAYOUT_HALF = _RELAYOUT_COLS // 2
_RELAYOUT_SHIFT = 14
_RELAYOUT_GRID = -(-VOCAB // _RELAYOUT_COLS)
VOCAB_LIN = _RELAYOUT_GRID * _RELAYOUT_COLS


def _table_relayout_body(tt_ref, out_ref):
    out_ref[:, :EMBED_DIM] = tt_ref[:, :_RELAYOUT_HALF].T
    out_ref[:, EMBED_DIM:] = tt_ref[:, _RELAYOUT_HALF:].T


_table_relayout_call = pl.pallas_call(
    _table_relayout_body,
    grid=(_RELAYOUT_GRID,),
    in_specs=[pl.BlockSpec((EMBED_DIM, _RELAYOUT_COLS), lambda i: (0, i))],
    out_specs=pl.BlockSpec((_RELAYOUT_HALF, 128), lambda i: (i, 0)),
    out_shape=jax.ShapeDtypeStruct((VOCAB_LIN // 2, 128), jnp.float32),
)


def kernel(indices, table, W, b):
    v = jnp.clip(indices.astype(jnp.int32).reshape(-1), 0, VOCAB - 1)
    i_blk = v >> _RELAYOUT_SHIFT
    r_off = v & (_RELAYOUT_COLS - 1)
    idx_flat = ((((i_blk << (_RELAYOUT_SHIFT - 1)) | (r_off & (_RELAYOUT_HALF - 1)))
                 << 1) | (r_off >> (_RELAYOUT_SHIFT - 1)))
    table_lin = _table_relayout_call(table.T).reshape(VOCAB_LIN, EMBED_DIM)
    pooled = _pool_call(idx_flat, table_lin)
    w_pad = jnp.zeros((EMBED_DIM, 128), jnp.float32).at[:, :OUTPUT_SIZE].set(W)
    b_pad = jnp.full((1, 128), -1e30, jnp.float32).at[0, :OUTPUT_SIZE].set(b)
    probs_pad = _dense_call(pooled, w_pad, b_pad)
    return probs_pad[:, :OUTPUT_SIZE]


# speedup vs baseline: 2.0493x; 1.0449x over previous
import functools

import jax
import jax.numpy as jnp
from jax import lax
from jax.experimental import pallas as pl
from jax.experimental.pallas import tpu as pltpu
from jax.experimental.pallas import tpu_sc as plsc

VOCAB = 1000000
EMBED_DIM = 64
MAXLEN = 200
BATCH = 4096
OUTPUT_SIZE = 10

NUM_CORES = 2
NUM_SUBCORES = 16
NUM_WORKERS = NUM_CORES * NUM_SUBCORES
ROWS_PER_WORKER = BATCH // NUM_WORKERS
CHUNK_A = 128
CHUNK_B = MAXLEN - CHUNK_A
LANES = 16
MAXLEN_PAD = 256
VECS_PER_ROW = EMBED_DIM // LANES
UNROLL = 4


def _pool_body(idx_hbm, table_hbm, out_hbm, idx_v, rows_v, pooled_v, sem0, sem1):
    wid = lax.axis_index("s") * NUM_CORES + lax.axis_index("c")
    rbase = wid * ROWS_PER_WORKER

    pltpu.sync_copy(idx_hbm.at[pl.ds(rbase * MAXLEN, ROWS_PER_WORKER * MAXLEN)],
                    idx_v)

    sems = (sem0, sem1)

    def issue(r, slot):
        pltpu.async_copy(table_hbm.at[idx_v.at[pl.ds(r * MAXLEN, CHUNK_A)]],
                         rows_v.at[slot, pl.ds(0, CHUNK_A)], sems[slot])
        pltpu.async_copy(table_hbm.at[idx_v.at[pl.ds(r * MAXLEN + CHUNK_A, CHUNK_B)]],
                         rows_v.at[slot, pl.ds(CHUNK_A, CHUNK_B)], sems[slot])

    def wait_slot(slot):
        pltpu.make_async_copy(table_hbm.at[pl.ds(0, MAXLEN), :],
                              rows_v.at[slot], sems[slot]).wait()

    def accum(r, slot):
        def step(i, acc):
            base = i * UNROLL
            acc = list(acc)
            for u in range(UNROLL):
                row = base + u
                for d in range(VECS_PER_ROW):
                    acc[d] = acc[d] + rows_v[slot, row, pl.ds(d * LANES, LANES)]
            return tuple(acc)

        zero = jnp.zeros((LANES,), jnp.float32)
        acc = lax.fori_loop(0, MAXLEN // UNROLL, step, (zero,) * VECS_PER_ROW)
        scale = jnp.float32(1.0 / MAXLEN)
        for d in range(VECS_PER_ROW):
            pooled_v[r, pl.ds(d * LANES, LANES)] = acc[d] * scale

    issue(0, 0)
    issue(1, 1)

    def outer(i, carry):
        r0 = 2 * i
        for slot in range(2):
            r = r0 + slot
            wait_slot(slot)
            accum(r, slot)

            @pl.when(r + 2 < ROWS_PER_WORKER)
            def _():
                issue(r + 2, slot)
        return carry

    lax.fori_loop(0, ROWS_PER_WORKER // 2, outer, 0)

    pltpu.sync_copy(pooled_v,
                    out_hbm.at[pl.ds(wid * ROWS_PER_WORKER, ROWS_PER_WORKER), :])


_pool_call = pl.kernel(
    _pool_body,
    out_type=jax.ShapeDtypeStruct((BATCH, EMBED_DIM), jnp.float32),
    mesh=plsc.VectorSubcoreMesh(core_axis_name="c", subcore_axis_name="s",
                                num_cores=NUM_CORES, num_subcores=NUM_SUBCORES),
    scratch_types=[
        pltpu.VMEM((ROWS_PER_WORKER * MAXLEN,), jnp.int32),
        pltpu.VMEM((2, MAXLEN, EMBED_DIM), jnp.float32),
        pltpu.VMEM((ROWS_PER_WORKER, EMBED_DIM), jnp.float32),
        pltpu.SemaphoreType.DMA,
        pltpu.SemaphoreType.DMA,
    ],
    compiler_params=pltpu.CompilerParams(use_tc_tiling_on_sc=False),
)


def _dense_softmax_body(pooled_ref, w_ref, b_ref, out_ref):
    logits = jnp.dot(pooled_ref[...], w_ref[...],
                     preferred_element_type=jnp.float32) + b_ref[...]
    m = jnp.max(logits, axis=-1, keepdims=True)
    e = jnp.exp(logits - m)
    out_ref[...] = e / jnp.sum(e, axis=-1, keepdims=True)


_dense_call = pl.pallas_call(
    _dense_softmax_body,
    out_shape=jax.ShapeDtypeStruct((BATCH, 128), jnp.float32),
)


_RELAYOUT_COLS = 32768
_RELAYOUT_HALF = _RELAYOUT_COLS // 2
_RELAYOUT_SHIFT = 15
_RELAYOUT_GRID = -(-VOCAB // _RELAYOUT_COLS)
VOCAB_LIN = _RELAYOUT_GRID * _RELAYOUT_COLS


def _table_relayout_body(tt_ref, out_ref):
    out_ref[:, :EMBED_DIM] = tt_ref[:, :_RELAYOUT_HALF].T
    out_ref[:, EMBED_DIM:] = tt_ref[:, _RELAYOUT_HALF:].T


_table_relayout_call = pl.pallas_call(
    _table_relayout_body,
    grid=(_RELAYOUT_GRID,),
    in_specs=[pl.BlockSpec((EMBED_DIM, _RELAYOUT_COLS), lambda i: (0, i))],
    out_specs=pl.BlockSpec((_RELAYOUT_HALF, 128), lambda i: (i, 0)),
    out_shape=jax.ShapeDtypeStruct((VOCAB_LIN // 2, 128), jnp.float32),
)


def kernel(indices, table, W, b):
    v = jnp.clip(indices.astype(jnp.int32).reshape(-1), 0, VOCAB - 1)
    i_blk = v >> _RELAYOUT_SHIFT
    r_off = v & (_RELAYOUT_COLS - 1)
    idx_flat = ((((i_blk << (_RELAYOUT_SHIFT - 1)) | (r_off & (_RELAYOUT_HALF - 1)))
                 << 1) | (r_off >> (_RELAYOUT_SHIFT - 1)))
    table_lin = _table_relayout_call(table.T).reshape(VOCAB_LIN, EMBED_DIM)
    pooled = _pool_call(idx_flat, table_lin)
    w_pad = jnp.zeros((EMBED_DIM, 128), jnp.float32).at[:, :OUTPUT_SIZE].set(W)
    b_pad = jnp.full((1, 128), -1e30, jnp.float32).at[0, :OUTPUT_SIZE].set(b)
    probs_pad = _dense_call(pooled, w_pad, b_pad)
    return probs_pad[:, :OUTPUT_SIZE]
